# Initial kernel scaffold; baseline (speedup 1.0000x reference)
import jax
import jax.numpy as jnp
from jax.experimental import pallas as pl


def kernel(x, te_W, te_b, pos_embed, mask_embed, dpe, pe_W1, pe_b1, pe_W2, pe_b2, e2d_W1, e2d_b1, e2d_W2, e2d_b2, d_W1, d_b1, d_W2, d_b2, d_W3, d_b3, head_W, head_b):
    raise NotImplementedError("write your pallas kernel here")


# speedup vs baseline: 9.0808x; 9.0808x over previous
import numpy as np
import jax
import jax.numpy as jnp
from jax.experimental import pallas as pl
from jax.experimental.pallas import tpu as pltpu

_TILE = 256
_MASK_RATIO = 0.75
_mask_cache = {}


def _threefry2x32(k0, k1, c0, c1):
    def rotl(v, r):
        return ((v << np.uint32(r)) | (v >> np.uint32(32 - r))).astype(np.uint32)

    rots = [[13, 15, 26, 6], [17, 29, 16, 24]]
    ks = [np.uint32(k0), np.uint32(k1),
          np.uint32(k0) ^ np.uint32(k1) ^ np.uint32(0x1BD11BDA)]
    x0 = (c0 + ks[0]).astype(np.uint32)
    x1 = (c1 + ks[1]).astype(np.uint32)
    for i in range(5):
        for r in rots[i % 2]:
            x0 = (x0 + x1).astype(np.uint32)
            x1 = rotl(x1, r) ^ x0
        x0 = (x0 + ks[(i + 1) % 3]).astype(np.uint32)
        x1 = (x1 + ks[(i + 2) % 3] + np.uint32(i + 1)).astype(np.uint32)
    return x0, x1


def _uniform01(seed, shape):
    n = int(np.prod(shape))
    idx = np.arange(n, dtype=np.uint64)
    c0 = (idx >> np.uint64(32)).astype(np.uint32)
    c1 = (idx & np.uint64(0xFFFFFFFF)).astype(np.uint32)
    k0 = np.uint32((seed >> 32) & 0xFFFFFFFF)
    k1 = np.uint32(seed & 0xFFFFFFFF)
    o0, o1 = _threefry2x32(k0, k1, c0, c1)
    bits = o0 ^ o1
    f = ((bits >> np.uint32(9)) | np.uint32(0x3F800000)).view(np.float32)
    return (f - np.float32(1.0)).reshape(shape)


def _masked_mask(B, N):
    if (B, N) not in _mask_cache:
        u = _uniform01(42, (B, N))
        shuffle = np.argsort(u, axis=-1, kind="stable")
        num_visible = int((1.0 - _MASK_RATIO) * N)
        num_masked = N - num_visible
        m = np.zeros((B, N), np.float32)
        m[np.arange(B)[:, None], shuffle[:, :num_masked]] = 1.0
        _mask_cache[(B, N)] = (m, num_masked)
    return _mask_cache[(B, N)]


def _leaky(v):
    return jnp.where(v >= 0, v, 0.01 * v)


def _body(dpe_ref, me_ref, w1_ref, b1_ref, w2_ref, b2_ref, w3_ref, b3_ref,
          wh_ref, bh_ref, xf_ref, mt_ref, recons_ref, acc_ref):
    i = pl.program_id(0)
    a0 = dpe_ref[...] + me_ref[...]
    h = _leaky(jnp.dot(a0, w1_ref[...], preferred_element_type=jnp.float32)
               + b1_ref[...])
    h = _leaky(jnp.dot(h, w2_ref[...], preferred_element_type=jnp.float32)
               + b2_ref[...])
    d3 = jnp.dot(h, w3_ref[...], preferred_element_type=jnp.float32) + b3_ref[...]
    pr = jnp.dot(d3, wh_ref[...], preferred_element_type=jnp.float32) + bh_ref[...]

    part = jnp.float32(0.0)
    for b in range(xf_ref.shape[0]):
        xb = xf_ref[b]
        m = mt_ref[:, b:b + 1]
        t = jnp.where(m > 0, pr - xb, 0.0)
        recons_ref[b] = xb + t
        part = part + jnp.sum(jnp.abs(t))

    @pl.when(i == 0)
    def _():
        acc_ref[0, 0] = 0.0

    acc_ref[0, 0] += part


def kernel(x, te_W, te_b, pos_embed, mask_embed, dpe, pe_W1, pe_b1, pe_W2,
           pe_b2, e2d_W1, e2d_b1, e2d_W2, e2d_b2, d_W1, d_b1, d_W2, d_b2,
           d_W3, d_b3, head_W, head_b):
    B, N = x.shape[0], x.shape[1]
    feat = int(np.prod(x.shape[2:]))
    latent = dpe.shape[1]
    mask_np, num_masked = _masked_mask(B, N)

    n_pad = ((N + _TILE - 1) // _TILE) * _TILE
    nt = n_pad // _TILE

    xf = x.reshape(B, N, feat)
    mt = np.zeros((n_pad, 8), np.float32)
    mt[:N, :B] = mask_np.T
    mt = jnp.asarray(mt)

    h1 = d_W1.shape[1]

    recons_p, acc = pl.pallas_call(
        _body,
        grid=(nt,),
        in_specs=[
            pl.BlockSpec((_TILE, latent), lambda i: (i, 0)),
            pl.BlockSpec((1, latent), lambda i: (0, 0)),
            pl.BlockSpec((latent, h1), lambda i: (0, 0)),
            pl.BlockSpec((1, h1), lambda i: (0, 0)),
            pl.BlockSpec((h1, latent), lambda i: (0, 0)),
            pl.BlockSpec((1, latent), lambda i: (0, 0)),
            pl.BlockSpec((latent, feat), lambda i: (0, 0)),
            pl.BlockSpec((1, feat), lambda i: (0, 0)),
            pl.BlockSpec((feat, feat), lambda i: (0, 0)),
            pl.BlockSpec((1, feat), lambda i: (0, 0)),
            pl.BlockSpec((B, _TILE, feat), lambda i: (0, i, 0)),
            pl.BlockSpec((_TILE, 8), lambda i: (i, 0)),
        ],
        out_specs=(
            pl.BlockSpec((B, _TILE, feat), lambda i: (0, i, 0)),
            pl.BlockSpec(memory_space=pltpu.SMEM),
        ),
        out_shape=(
            jax.ShapeDtypeStruct((B, N, feat), jnp.float32),
            jax.ShapeDtypeStruct((1, 1), jnp.float32),
        ),
    )(dpe, mask_embed[None, :], d_W1, d_b1[None, :], d_W2, d_b2[None, :],
      d_W3, d_b3[None, :], head_W, head_b[None, :], xf, mt)

    recons = recons_p.reshape(x.shape)
    loss = acc[0, 0] / np.float32(B * num_masked * feat)
    return (recons, loss)
